# baseline (speedup 1.0000x reference)
import functools

import jax
import jax.numpy as jnp
from jax import lax
from jax.experimental import pallas as pl
from jax.experimental.pallas import tpu as pltpu
from jax.experimental.pallas import tpu_sc as plsc

N = 10000
D = 128
F = 16
NP = 10240
NC = 2
NS = 16
NW = NC * NS
CH = 128
NG = 16
RPS = NP // NS
ROWBLK = 1280


def _sc_degree_body(dst_hbm, out_hbm, dst_v, ones_v, zbuf, deg_sh):
  c = lax.axis_index("c")
  s = lax.axis_index("s")
  wid = s * NC + c
  nch = dst_v.shape[0]
  pltpu.sync_copy(dst_hbm.at[wid], dst_v)
  ov = jnp.ones((16,), jnp.float32)
  for k in range(CH // 16):
    ones_v[pl.ds(k * 16, 16)] = ov
  zv = jnp.zeros((16,), jnp.float32)
  for k in range(RPS // 16):
    zbuf[pl.ds(k * 16, 16)] = zv
  pltpu.sync_copy(zbuf, deg_sh.at[pl.ds(s * RPS, RPS)])
  plsc.subcore_barrier()

  def body(j, _):
    pltpu.sync_copy(ones_v, deg_sh.at[dst_v.at[j]], add=True)
    return _

  lax.fori_loop(0, nch, body, None)
  plsc.subcore_barrier()

  @pl.when(s == 0)
  def _():
    pltpu.sync_copy(deg_sh, out_hbm.at[c])


def _sc_scatter_body(table_hbm, src_hbm, dst_hbm, out_hbm,
                     src_v, dst_v, rows_a, rows_b, acc, sem_a, sem_b):
  c = lax.axis_index("c")
  s = lax.axis_index("s")
  wid = s * NC + c
  g = lax.rem(wid, NG)
  f = wid // NG
  nch = src_v.shape[0]
  pltpu.sync_copy(src_hbm.at[g], src_v)
  pltpu.sync_copy(dst_hbm.at[g], dst_v)

  zv = jnp.zeros((16,), jnp.float32)

  def zbody(i, carry):
    acc[pl.ds(i * 16, 16)] = zv
    return carry

  lax.fori_loop(0, NP * 8 // 16, zbody, None)

  iota = lax.iota(jnp.int32, 16)
  pair = lax.shift_right_logical(iota, 3)
  c8 = iota & 7
  m_lo = iota < 8
  m_hi = iota >= 8

  def sbody(j, carry):
    for b in range(CH // 16):
      sl = pl.ds(b * 16, 16)
      src_v[j, sl] = src_v[j, sl] * 2 + f
    return carry

  lax.fori_loop(0, nch, sbody, None)

  def chunk_compute(j, rb):
    jv = jnp.full((16,), j, jnp.int32)
    for b in range(CH // 16):
      for q in range(8):
        dp = plsc.load_gather(dst_v, [jv, pair + (16 * b + 2 * q)])
        vals = plsc.load_gather(rb, [pair + (16 * b + 2 * q), c8])
        addr = dp * 8 + c8
        plsc.addupdate_scatter(acc, [addr], vals, mask=m_lo)
        plsc.addupdate_scatter(acc, [addr], vals, mask=m_hi)

  cp0 = pltpu.async_copy(table_hbm.at[src_v.at[0]], rows_a, sem_a)

  def body(j, carry):
    ja = 2 * j
    pltpu.async_copy(table_hbm.at[src_v.at[ja + 1]], rows_b, sem_b)
    pltpu.make_async_copy(table_hbm.at[src_v.at[ja]], rows_a, sem_a).wait()
    chunk_compute(ja, rows_a)

    @pl.when(ja + 2 < nch)
    def _():
      pltpu.async_copy(table_hbm.at[src_v.at[ja + 2]], rows_a, sem_a)

    pltpu.make_async_copy(table_hbm.at[src_v.at[ja + 1]], rows_b, sem_b).wait()
    chunk_compute(ja + 1, rows_b)
    return carry

  lax.fori_loop(0, nch // 2, body, None)
  pltpu.sync_copy(acc, out_hbm.at[wid])


def _sc_degree(dst3, nch):
  mesh = plsc.VectorSubcoreMesh(core_axis_name="c", subcore_axis_name="s",
                                num_cores=NC, num_subcores=NS)
  fn = pl.kernel(
      _sc_degree_body,
      out_type=jax.ShapeDtypeStruct((NC, NP), jnp.float32),
      mesh=mesh,
      scratch_types=[
          pltpu.VMEM((nch, CH), jnp.int32),
          pltpu.VMEM((CH,), jnp.float32),
          pltpu.VMEM((RPS,), jnp.float32),
          pltpu.VMEM_SHARED((NP,), jnp.float32),
      ],
  )
  return fn(dst3)


def _sc_scatter(table, src3, dst3, nch):
  mesh = plsc.VectorSubcoreMesh(core_axis_name="c", subcore_axis_name="s",
                                num_cores=NC, num_subcores=NS)
  fn = pl.kernel(
      _sc_scatter_body,
      out_type=jax.ShapeDtypeStruct((NW, NP * 8), jnp.float32),
      mesh=mesh,
      compiler_params=pltpu.CompilerParams(use_tc_tiling_on_sc=False,
                                           needs_layout_passes=False),
      scratch_types=[
          pltpu.VMEM((nch, CH), jnp.int32),
          pltpu.VMEM((nch, CH), jnp.int32),
          pltpu.VMEM((CH, 8), jnp.float32),
          pltpu.VMEM((CH, 8), jnp.float32),
          pltpu.VMEM((NP * 8,), jnp.float32),
          pltpu.SemaphoreType.DMA,
          pltpu.SemaphoreType.DMA,
      ],
  )
  return fn(table, src3, dst3)


def _tc1_body(x_ref, w_ref, degp_ref, g_ref, dinv_ref):
  deg = degp_ref[0] + degp_ref[1] + 1.0
  dinv = lax.rsqrt(deg)
  h = jnp.dot(x_ref[...], w_ref[...], preferred_element_type=jnp.float32)
  dinv_ref[...] = dinv
  g_ref[...] = dinv * h


def _merge_partials(acc_ref):
  a = acc_ref[...]
  s0 = jnp.sum(a[0:NG], axis=0)
  s1 = jnp.sum(a[NG:NW], axis=0)
  return jnp.concatenate([s0, s1], axis=-1)


def _tc2_body(acc_ref, g_ref, dinv_ref, b_ref, w_ref, g2_ref):
  dinv = dinv_ref[...]
  z = dinv * (_merge_partials(acc_ref) + g_ref[...]) + b_ref[...]
  o = jnp.maximum(z, 0.0)
  h2 = jnp.dot(o, w_ref[...], preferred_element_type=jnp.float32)
  g2_ref[...] = dinv * h2


def _tc3_body(acc_ref, g_ref, dinv_ref, b_ref, out_ref):
  z = dinv_ref[...] * (_merge_partials(acc_ref) + g_ref[...]) + b_ref[...]
  m = jnp.max(z, axis=1, keepdims=True)
  e = jnp.exp(z - m)
  lse = jnp.log(jnp.sum(e, axis=1, keepdims=True))
  out_ref[...] = z - m - lse


def _tc1(xp, W1, degp):
  grid = (NP // ROWBLK,)
  return pl.pallas_call(
      _tc1_body,
      grid=grid,
      in_specs=[
          pl.BlockSpec((ROWBLK, D), lambda i: (i, 0)),
          pl.BlockSpec((D, F), lambda i: (0, 0)),
          pl.BlockSpec((2, ROWBLK, 1), lambda i: (0, i, 0)),
      ],
      out_specs=[
          pl.BlockSpec((ROWBLK, F), lambda i: (i, 0)),
          pl.BlockSpec((ROWBLK, 1), lambda i: (i, 0)),
      ],
      out_shape=[
          jax.ShapeDtypeStruct((NP, F), jnp.float32),
          jax.ShapeDtypeStruct((NP, 1), jnp.float32),
      ],
  )(xp, W1, degp)


def _tc2(acc1, g1, dinv, b1, W2):
  grid = (NP // ROWBLK,)
  return pl.pallas_call(
      _tc2_body,
      grid=grid,
      in_specs=[
          pl.BlockSpec((NW, ROWBLK, 8), lambda i: (0, i, 0)),
          pl.BlockSpec((ROWBLK, F), lambda i: (i, 0)),
          pl.BlockSpec((ROWBLK, 1), lambda i: (i, 0)),
          pl.BlockSpec((F,), lambda i: (0,)),
          pl.BlockSpec((F, F), lambda i: (0, 0)),
      ],
      out_specs=pl.BlockSpec((ROWBLK, F), lambda i: (i, 0)),
      out_shape=jax.ShapeDtypeStruct((NP, F), jnp.float32),
  )(acc1, g1, dinv, b1, W2)


def _tc3(acc2, g2, dinv, b2):
  grid = (NP // ROWBLK,)
  return pl.pallas_call(
      _tc3_body,
      grid=grid,
      in_specs=[
          pl.BlockSpec((NW, ROWBLK, 8), lambda i: (0, i, 0)),
          pl.BlockSpec((ROWBLK, F), lambda i: (i, 0)),
          pl.BlockSpec((ROWBLK, 1), lambda i: (i, 0)),
          pl.BlockSpec((F,), lambda i: (0,)),
      ],
      out_specs=pl.BlockSpec((ROWBLK, F), lambda i: (i, 0)),
      out_shape=jax.ShapeDtypeStruct((NP, F), jnp.float32),
  )(acc2, g2, dinv, b2)


def kernel(x, edge_index, W1, b1, W2, b2):
  src = edge_index[0].astype(jnp.int32)
  dst = edge_index[1].astype(jnp.int32)
  e = src.shape[0]
  per = NW * CH
  nchd = -(-e // per)
  ep = nchd * per
  nchg = 2 * nchd
  src_p = jnp.concatenate([src, jnp.zeros((ep - e,), jnp.int32)])
  dst_p = jnp.concatenate([dst, jnp.full((ep - e,), N, jnp.int32)])
  dst3d = dst_p.reshape(NW, nchd, CH)
  src3g = src_p.reshape(NG, nchg, CH)
  dst3g = dst_p.reshape(NG, nchg, CH)

  xp = jnp.pad(x.astype(jnp.float32), ((0, NP - N), (0, 0)))
  degp = _sc_degree(dst3d, nchd)
  g1, dinv = _tc1(xp, W1, degp.reshape(NC, NP, 1))
  acc1 = _sc_scatter(g1.reshape(2 * NP, 8), src3g, dst3g, nchg)
  g2 = _tc2(acc1.reshape(NW, NP, 8), g1, dinv, b1, W2)
  acc2 = _sc_scatter(g2.reshape(2 * NP, 8), src3g, dst3g, nchg)
  out = _tc3(acc2.reshape(NW, NP, 8), g2, dinv, b2)
  return out[:N]


# speedup vs baseline: 1.0559x; 1.0559x over previous
import functools

import jax
import jax.numpy as jnp
from jax import lax
from jax.experimental import pallas as pl
from jax.experimental.pallas import tpu as pltpu
from jax.experimental.pallas import tpu_sc as plsc

N = 10000
D = 128
F = 16
NP = 10240
NC = 2
NS = 16
NW = NC * NS
CH = 128
NG = 16
RPS = NP // NS
ROWBLK = 1280


def _sc_degree_body(dst_hbm, out_hbm, dst_v, ones_v, zbuf, deg_sh):
  c = lax.axis_index("c")
  s = lax.axis_index("s")
  wid = s * NC + c
  nch = dst_v.shape[0]
  pltpu.sync_copy(dst_hbm.at[wid], dst_v)
  ov = jnp.ones((16,), jnp.float32)
  for k in range(CH // 16):
    ones_v[pl.ds(k * 16, 16)] = ov
  zv = jnp.zeros((16,), jnp.float32)
  for k in range(RPS // 16):
    zbuf[pl.ds(k * 16, 16)] = zv
  pltpu.sync_copy(zbuf, deg_sh.at[pl.ds(s * RPS, RPS)])
  plsc.subcore_barrier()

  def body(j, _):
    pltpu.sync_copy(ones_v, deg_sh.at[dst_v.at[j]], add=True)
    return _

  lax.fori_loop(0, nch, body, None)
  plsc.subcore_barrier()

  @pl.when(s == 0)
  def _():
    pltpu.sync_copy(deg_sh, out_hbm.at[c])


def _sc_scatter_body(table_hbm, src_hbm, dst_hbm, out_hbm,
                     src_v, dst_v, rows_a, rows_b, acc, sem_a, sem_b):
  c = lax.axis_index("c")
  s = lax.axis_index("s")
  wid = s * NC + c
  g = lax.rem(wid, NG)
  f = wid // NG
  nch = src_v.shape[0]
  pltpu.sync_copy(src_hbm.at[g], src_v)
  pltpu.sync_copy(dst_hbm.at[g], dst_v)

  zv = jnp.zeros((16,), jnp.float32)

  def zbody(i, carry):
    acc[pl.ds(i * 16, 16)] = zv
    return carry

  lax.fori_loop(0, NP * 8 // 16, zbody, None)

  iota = lax.iota(jnp.int32, 16)
  pair = lax.shift_right_logical(iota, 3)
  c8 = iota & 7
  m_lo = iota < 8
  m_hi = iota >= 8

  def sbody(j, carry):
    for b in range(CH // 16):
      sl = pl.ds(b * 16, 16)
      src_v[j, sl] = src_v[j, sl] * 2 + f
    return carry

  lax.fori_loop(0, nch, sbody, None)

  def chunk_compute(j, rb):
    jv = jnp.full((16,), j, jnp.int32)
    for b in range(CH // 16):
      dps = [plsc.load_gather(dst_v, [jv, pair + (16 * b + 2 * q)])
             for q in range(8)]
      valss = [plsc.load_gather(rb, [pair + (16 * b + 2 * q), c8])
               for q in range(8)]
      for q in range(8):
        addr = dps[q] * 8 + c8
        plsc.addupdate_scatter(acc, [addr], valss[q], mask=m_lo)
        plsc.addupdate_scatter(acc, [addr], valss[q], mask=m_hi)

  cp0 = pltpu.async_copy(table_hbm.at[src_v.at[0]], rows_a, sem_a)

  def body(j, carry):
    ja = 2 * j
    pltpu.async_copy(table_hbm.at[src_v.at[ja + 1]], rows_b, sem_b)
    pltpu.make_async_copy(table_hbm.at[src_v.at[ja]], rows_a, sem_a).wait()
    chunk_compute(ja, rows_a)

    @pl.when(ja + 2 < nch)
    def _():
      pltpu.async_copy(table_hbm.at[src_v.at[ja + 2]], rows_a, sem_a)

    pltpu.make_async_copy(table_hbm.at[src_v.at[ja + 1]], rows_b, sem_b).wait()
    chunk_compute(ja + 1, rows_b)
    return carry

  lax.fori_loop(0, nch // 2, body, None)
  pltpu.sync_copy(acc, out_hbm.at[wid])


def _sc_degree(dst3, nch):
  mesh = plsc.VectorSubcoreMesh(core_axis_name="c", subcore_axis_name="s",
                                num_cores=NC, num_subcores=NS)
  fn = pl.kernel(
      _sc_degree_body,
      out_type=jax.ShapeDtypeStruct((NC, NP), jnp.float32),
      mesh=mesh,
      scratch_types=[
          pltpu.VMEM((nch, CH), jnp.int32),
          pltpu.VMEM((CH,), jnp.float32),
          pltpu.VMEM((RPS,), jnp.float32),
          pltpu.VMEM_SHARED((NP,), jnp.float32),
      ],
  )
  return fn(dst3)


def _sc_scatter(table, src3, dst3, nch):
  mesh = plsc.VectorSubcoreMesh(core_axis_name="c", subcore_axis_name="s",
                                num_cores=NC, num_subcores=NS)
  fn = pl.kernel(
      _sc_scatter_body,
      out_type=jax.ShapeDtypeStruct((NW, NP * 8), jnp.float32),
      mesh=mesh,
      compiler_params=pltpu.CompilerParams(use_tc_tiling_on_sc=False,
                                           needs_layout_passes=False),
      scratch_types=[
          pltpu.VMEM((nch, CH), jnp.int32),
          pltpu.VMEM((nch, CH), jnp.int32),
          pltpu.VMEM((CH, 8), jnp.float32),
          pltpu.VMEM((CH, 8), jnp.float32),
          pltpu.VMEM((NP * 8,), jnp.float32),
          pltpu.SemaphoreType.DMA,
          pltpu.SemaphoreType.DMA,
      ],
  )
  return fn(table, src3, dst3)


def _tc1_body(x_ref, w_ref, degp_ref, g_ref, dinv_ref):
  deg = degp_ref[0] + degp_ref[1] + 1.0
  dinv = lax.rsqrt(deg)
  h = jnp.dot(x_ref[...], w_ref[...], preferred_element_type=jnp.float32)
  dinv_ref[...] = dinv
  g_ref[...] = dinv * h


def _merge_partials(acc_ref):
  a = acc_ref[...]
  s0 = jnp.sum(a[0:NG], axis=0)
  s1 = jnp.sum(a[NG:NW], axis=0)
  return jnp.concatenate([s0, s1], axis=-1)


def _tc2_body(acc_ref, g_ref, dinv_ref, b_ref, w_ref, g2_ref):
  dinv = dinv_ref[...]
  z = dinv * (_merge_partials(acc_ref) + g_ref[...]) + b_ref[...]
  o = jnp.maximum(z, 0.0)
  h2 = jnp.dot(o, w_ref[...], preferred_element_type=jnp.float32)
  g2_ref[...] = dinv * h2


def _tc3_body(acc_ref, g_ref, dinv_ref, b_ref, out_ref):
  z = dinv_ref[...] * (_merge_partials(acc_ref) + g_ref[...]) + b_ref[...]
  m = jnp.max(z, axis=1, keepdims=True)
  e = jnp.exp(z - m)
  lse = jnp.log(jnp.sum(e, axis=1, keepdims=True))
  out_ref[...] = z - m - lse


def _tc1(xp, W1, degp):
  grid = (NP // ROWBLK,)
  return pl.pallas_call(
      _tc1_body,
      grid=grid,
      in_specs=[
          pl.BlockSpec((ROWBLK, D), lambda i: (i, 0)),
          pl.BlockSpec((D, F), lambda i: (0, 0)),
          pl.BlockSpec((2, ROWBLK, 1), lambda i: (0, i, 0)),
      ],
      out_specs=[
          pl.BlockSpec((ROWBLK, F), lambda i: (i, 0)),
          pl.BlockSpec((ROWBLK, 1), lambda i: (i, 0)),
      ],
      out_shape=[
          jax.ShapeDtypeStruct((NP, F), jnp.float32),
          jax.ShapeDtypeStruct((NP, 1), jnp.float32),
      ],
  )(xp, W1, degp)


def _tc2(acc1, g1, dinv, b1, W2):
  grid = (NP // ROWBLK,)
  return pl.pallas_call(
      _tc2_body,
      grid=grid,
      in_specs=[
          pl.BlockSpec((NW, ROWBLK, 8), lambda i: (0, i, 0)),
          pl.BlockSpec((ROWBLK, F), lambda i: (i, 0)),
          pl.BlockSpec((ROWBLK, 1), lambda i: (i, 0)),
          pl.BlockSpec((F,), lambda i: (0,)),
          pl.BlockSpec((F, F), lambda i: (0, 0)),
      ],
      out_specs=pl.BlockSpec((ROWBLK, F), lambda i: (i, 0)),
      out_shape=jax.ShapeDtypeStruct((NP, F), jnp.float32),
  )(acc1, g1, dinv, b1, W2)


def _tc3(acc2, g2, dinv, b2):
  grid = (NP // ROWBLK,)
  return pl.pallas_call(
      _tc3_body,
      grid=grid,
      in_specs=[
          pl.BlockSpec((NW, ROWBLK, 8), lambda i: (0, i, 0)),
          pl.BlockSpec((ROWBLK, F), lambda i: (i, 0)),
          pl.BlockSpec((ROWBLK, 1), lambda i: (i, 0)),
          pl.BlockSpec((F,), lambda i: (0,)),
      ],
      out_specs=pl.BlockSpec((ROWBLK, F), lambda i: (i, 0)),
      out_shape=jax.ShapeDtypeStruct((NP, F), jnp.float32),
  )(acc2, g2, dinv, b2)


def kernel(x, edge_index, W1, b1, W2, b2):
  src = edge_index[0].astype(jnp.int32)
  dst = edge_index[1].astype(jnp.int32)
  e = src.shape[0]
  per = NW * CH
  nchd = -(-e // per)
  ep = nchd * per
  nchg = 2 * nchd
  src_p = jnp.concatenate([src, jnp.zeros((ep - e,), jnp.int32)])
  dst_p = jnp.concatenate([dst, jnp.full((ep - e,), N, jnp.int32)])
  dst3d = dst_p.reshape(NW, nchd, CH)
  src3g = src_p.reshape(NG, nchg, CH)
  dst3g = dst_p.reshape(NG, nchg, CH)

  xp = jnp.pad(x.astype(jnp.float32), ((0, NP - N), (0, 0)))
  degp = _sc_degree(dst3d, nchd)
  g1, dinv = _tc1(xp, W1, degp.reshape(NC, NP, 1))
  acc1 = _sc_scatter(g1.reshape(2 * NP, 8), src3g, dst3g, nchg)
  g2 = _tc2(acc1.reshape(NW, NP, 8), g1, dinv, b1, W2)
  acc2 = _sc_scatter(g2.reshape(2 * NP, 8), src3g, dst3g, nchg)
  out = _tc3(acc2.reshape(NW, NP, 8), g2, dinv, b2)
  return out[:N]


# speedup vs baseline: 2.8130x; 2.6640x over previous
import functools

import jax
import jax.numpy as jnp
from jax import lax
from jax.experimental import pallas as pl
from jax.experimental.pallas import tpu as pltpu
from jax.experimental.pallas import tpu_sc as plsc

N = 10000
D = 128
F = 16
NP = 10240
NC = 2
NS = 16
NW = NC * NS
CH = 128
NG = 16
RPS = NP // NS
ZR = 80
ROWBLK = 1280


def _sc_degree_body(dst_hbm, out_hbm, dst_v, ones_v, zbuf, deg_sh):
  c = lax.axis_index("c")
  s = lax.axis_index("s")
  wid = s * NC + c
  nch = dst_v.shape[0]
  pltpu.sync_copy(dst_hbm.at[wid], dst_v)
  ov = jnp.ones((16,), jnp.float32)
  for k in range(CH // 16):
    ones_v[pl.ds(k * 16, 16)] = ov
  zv = jnp.zeros((16,), jnp.float32)
  for k in range(RPS // 16):
    zbuf[pl.ds(k * 16, 16)] = zv
  pltpu.sync_copy(zbuf, deg_sh.at[pl.ds(s * RPS, RPS)])
  plsc.subcore_barrier()

  def body(j, _):
    pltpu.sync_copy(ones_v, deg_sh.at[dst_v.at[j]], add=True)
    return _

  lax.fori_loop(0, nch, body, None)
  plsc.subcore_barrier()

  @pl.when(s == 0)
  def _():
    pltpu.sync_copy(deg_sh, out_hbm.at[c])


def _sc_scatter_body(table_hbm, src_hbm, dst_hbm, out_hbm,
                     src_v, dst_v, rows, zbuf, acc_sh, gsems, ssems):
  c = lax.axis_index("c")
  s = lax.axis_index("s")
  wid = s * NC + c
  nch = src_v.shape[0]
  pltpu.sync_copy(src_hbm.at[wid], src_v)
  pltpu.sync_copy(dst_hbm.at[wid], dst_v)
  zv = jnp.zeros((16,), jnp.float32)
  for k in range(ZR):
    zbuf[k] = zv
  for k in range(RPS // ZR):
    pltpu.sync_copy(zbuf, acc_sh.at[pl.ds(s * RPS + k * ZR, ZR)])
  plsc.subcore_barrier()

  nbuf = 4
  for b in range(nbuf):
    pltpu.async_copy(table_hbm.at[src_v.at[b]], rows.at[b], gsems[b])

  def body(j, carry):
    for b in range(nbuf):
      jj = nbuf * j + b
      pltpu.make_async_copy(table_hbm.at[src_v.at[jj]], rows.at[b],
                            gsems[b]).wait()
      @pl.when(jj >= nbuf)
      def _():
        pltpu.make_async_copy(rows.at[b], acc_sh.at[dst_v.at[jj - nbuf]],
                              ssems[b]).wait()
      pltpu.async_copy(rows.at[b], acc_sh.at[dst_v.at[jj]], ssems[b],
                       add=True)
      @pl.when(jj + nbuf < nch)
      def _():
        pltpu.async_copy(table_hbm.at[src_v.at[jj + nbuf]], rows.at[b],
                         gsems[b])
    return carry

  lax.fori_loop(0, nch // nbuf, body, None)
  for b in range(nbuf):
    pltpu.make_async_copy(rows.at[b], acc_sh.at[dst_v.at[0]],
                          ssems[b]).wait()
  plsc.subcore_barrier()

  @pl.when(s == 0)
  def _():
    pltpu.sync_copy(acc_sh, out_hbm.at[c])


def _sc_degree(dst3, nch):
  mesh = plsc.VectorSubcoreMesh(core_axis_name="c", subcore_axis_name="s",
                                num_cores=NC, num_subcores=NS)
  fn = pl.kernel(
      _sc_degree_body,
      out_type=jax.ShapeDtypeStruct((NC, NP), jnp.float32),
      mesh=mesh,
      scratch_types=[
          pltpu.VMEM((nch, CH), jnp.int32),
          pltpu.VMEM((CH,), jnp.float32),
          pltpu.VMEM((RPS,), jnp.float32),
          pltpu.VMEM_SHARED((NP,), jnp.float32),
      ],
  )
  return fn(dst3)


def _sc_scatter(table, src3, dst3, nch):
  mesh = plsc.VectorSubcoreMesh(core_axis_name="c", subcore_axis_name="s",
                                num_cores=NC, num_subcores=NS)
  fn = pl.kernel(
      _sc_scatter_body,
      out_type=jax.ShapeDtypeStruct((NC, NP, F), jnp.float32),
      mesh=mesh,
      compiler_params=pltpu.CompilerParams(use_tc_tiling_on_sc=False),
      scratch_types=[
          pltpu.VMEM((nch, CH), jnp.int32),
          pltpu.VMEM((nch, CH), jnp.int32),
          pltpu.VMEM((4, CH, F), jnp.float32),
          pltpu.VMEM((ZR, F), jnp.float32),
          pltpu.VMEM_SHARED((NP, F), jnp.float32),
          [pltpu.SemaphoreType.DMA] * 4,
          [pltpu.SemaphoreType.DMA] * 4,
      ],
  )
  return fn(table, src3, dst3)


def _tc1_body(x_ref, w_ref, degp_ref, g_ref, dinv_ref):
  deg = degp_ref[0] + degp_ref[1] + 1.0
  dinv = lax.rsqrt(deg)
  h = jnp.dot(x_ref[...], w_ref[...], preferred_element_type=jnp.float32)
  dinv_ref[...] = dinv
  g_ref[...] = dinv * h


def _merge_partials(acc_ref):
  return acc_ref[0] + acc_ref[1]


def _tc2_body(acc_ref, g_ref, dinv_ref, b_ref, w_ref, g2_ref):
  dinv = dinv_ref[...]
  z = dinv * (_merge_partials(acc_ref) + g_ref[...]) + b_ref[...]
  o = jnp.maximum(z, 0.0)
  h2 = jnp.dot(o, w_ref[...], preferred_element_type=jnp.float32)
  g2_ref[...] = dinv * h2


def _tc3_body(acc_ref, g_ref, dinv_ref, b_ref, out_ref):
  z = dinv_ref[...] * (_merge_partials(acc_ref) + g_ref[...]) + b_ref[...]
  m = jnp.max(z, axis=1, keepdims=True)
  e = jnp.exp(z - m)
  lse = jnp.log(jnp.sum(e, axis=1, keepdims=True))
  out_ref[...] = z - m - lse


def _tc1(xp, W1, degp):
  grid = (NP // ROWBLK,)
  return pl.pallas_call(
      _tc1_body,
      grid=grid,
      in_specs=[
          pl.BlockSpec((ROWBLK, D), lambda i: (i, 0)),
          pl.BlockSpec((D, F), lambda i: (0, 0)),
          pl.BlockSpec((2, ROWBLK, 1), lambda i: (0, i, 0)),
      ],
      out_specs=[
          pl.BlockSpec((ROWBLK, F), lambda i: (i, 0)),
          pl.BlockSpec((ROWBLK, 1), lambda i: (i, 0)),
      ],
      out_shape=[
          jax.ShapeDtypeStruct((NP, F), jnp.float32),
          jax.ShapeDtypeStruct((NP, 1), jnp.float32),
      ],
  )(xp, W1, degp)


def _tc2(acc1, g1, dinv, b1, W2):
  grid = (NP // ROWBLK,)
  return pl.pallas_call(
      _tc2_body,
      grid=grid,
      in_specs=[
          pl.BlockSpec((NC, ROWBLK, F), lambda i: (0, i, 0)),
          pl.BlockSpec((ROWBLK, F), lambda i: (i, 0)),
          pl.BlockSpec((ROWBLK, 1), lambda i: (i, 0)),
          pl.BlockSpec((F,), lambda i: (0,)),
          pl.BlockSpec((F, F), lambda i: (0, 0)),
      ],
      out_specs=pl.BlockSpec((ROWBLK, F), lambda i: (i, 0)),
      out_shape=jax.ShapeDtypeStruct((NP, F), jnp.float32),
  )(acc1, g1, dinv, b1, W2)


def _tc3(acc2, g2, dinv, b2):
  grid = (NP // ROWBLK,)
  return pl.pallas_call(
      _tc3_body,
      grid=grid,
      in_specs=[
          pl.BlockSpec((NC, ROWBLK, F), lambda i: (0, i, 0)),
          pl.BlockSpec((ROWBLK, F), lambda i: (i, 0)),
          pl.BlockSpec((ROWBLK, 1), lambda i: (i, 0)),
          pl.BlockSpec((F,), lambda i: (0,)),
      ],
      out_specs=pl.BlockSpec((ROWBLK, F), lambda i: (i, 0)),
      out_shape=jax.ShapeDtypeStruct((NP, F), jnp.float32),
  )(acc2, g2, dinv, b2)


def kernel(x, edge_index, W1, b1, W2, b2):
  src = edge_index[0].astype(jnp.int32)
  dst = edge_index[1].astype(jnp.int32)
  e = src.shape[0]
  per = NW * CH
  nch = 4 * (-(-e // (4 * per)))
  ep = nch * per
  src_p = jnp.concatenate([src, jnp.zeros((ep - e,), jnp.int32)])
  dst_p = jnp.concatenate([dst, jnp.full((ep - e,), N, jnp.int32)])
  src3 = src_p.reshape(NW, nch, CH)
  dst3 = dst_p.reshape(NW, nch, CH)

  xp = jnp.pad(x.astype(jnp.float32), ((0, NP - N), (0, 0)))
  degp = _sc_degree(dst3, nch)
  g1, dinv = _tc1(xp, W1, degp.reshape(NC, NP, 1))
  acc1 = _sc_scatter(g1, src3, dst3, nch)
  g2 = _tc2(acc1, g1, dinv, b1, W2)
  acc2 = _sc_scatter(g2, src3, dst3, nch)
  out = _tc3(acc2, g2, dinv, b2)
  return out[:N]


# speedup vs baseline: 2.8142x; 1.0004x over previous
import functools

import jax
import jax.numpy as jnp
from jax import lax
from jax.experimental import pallas as pl
from jax.experimental.pallas import tpu as pltpu
from jax.experimental.pallas import tpu_sc as plsc

N = 10000
D = 128
F = 16
NP = 10240
NC = 2
NS = 16
NW = NC * NS
CH = 128
NG = 16
RPS = NP // NS
ZR = 80
ROWBLK = 1280


def _sc_degree_body(dst_hbm, out_hbm, dst_v, ones_v, zbuf, deg_sh, dsem):
  c = lax.axis_index("c")
  s = lax.axis_index("s")
  wid = s * NC + c
  nch = dst_v.shape[0]
  pltpu.sync_copy(dst_hbm.at[wid], dst_v)
  ov = jnp.ones((16,), jnp.float32)
  for k in range(CH // 16):
    ones_v[pl.ds(k * 16, 16)] = ov
  zv = jnp.zeros((16,), jnp.float32)
  for k in range(RPS // 16):
    zbuf[pl.ds(k * 16, 16)] = zv
  pltpu.sync_copy(zbuf, deg_sh.at[pl.ds(s * RPS, RPS)])
  plsc.subcore_barrier()

  def body(j, _):
    pltpu.async_copy(ones_v, deg_sh.at[dst_v.at[j]], dsem, add=True)
    return _

  lax.fori_loop(0, nch, body, None)

  def drain(j, _):
    pltpu.make_async_copy(ones_v, deg_sh.at[dst_v.at[0]], dsem).wait()
    return _

  lax.fori_loop(0, nch, drain, None)
  plsc.subcore_barrier()

  @pl.when(s == 0)
  def _():
    pltpu.sync_copy(deg_sh, out_hbm.at[c])


def _sc_scatter_body(table_hbm, src_hbm, dst_hbm, out_hbm,
                     src_v, dst_v, rows, zbuf, acc_sh, gsems, ssems):
  c = lax.axis_index("c")
  s = lax.axis_index("s")
  wid = s * NC + c
  nch = src_v.shape[0]
  pltpu.sync_copy(src_hbm.at[wid], src_v)
  pltpu.sync_copy(dst_hbm.at[wid], dst_v)
  zv = jnp.zeros((16,), jnp.float32)
  for k in range(ZR):
    zbuf[k] = zv
  for k in range(RPS // ZR):
    pltpu.sync_copy(zbuf, acc_sh.at[pl.ds(s * RPS + k * ZR, ZR)])
  plsc.subcore_barrier()

  nbuf = 4
  for b in range(nbuf):
    pltpu.async_copy(table_hbm.at[src_v.at[b]], rows.at[b], gsems[b])

  def body(j, carry):
    for b in range(nbuf):
      jj = nbuf * j + b
      pltpu.make_async_copy(table_hbm.at[src_v.at[jj]], rows.at[b],
                            gsems[b]).wait()
      @pl.when(jj >= nbuf)
      def _():
        pltpu.make_async_copy(rows.at[b], acc_sh.at[dst_v.at[jj - nbuf]],
                              ssems[b]).wait()
      pltpu.async_copy(rows.at[b], acc_sh.at[dst_v.at[jj]], ssems[b],
                       add=True)
      @pl.when(jj + nbuf < nch)
      def _():
        pltpu.async_copy(table_hbm.at[src_v.at[jj + nbuf]], rows.at[b],
                         gsems[b])
    return carry

  lax.fori_loop(0, nch // nbuf, body, None)
  for b in range(nbuf):
    pltpu.make_async_copy(rows.at[b], acc_sh.at[dst_v.at[0]],
                          ssems[b]).wait()
  plsc.subcore_barrier()

  @pl.when(s == 0)
  def _():
    pltpu.sync_copy(acc_sh, out_hbm.at[c])


def _sc_degree(dst3, nch):
  mesh = plsc.VectorSubcoreMesh(core_axis_name="c", subcore_axis_name="s",
                                num_cores=NC, num_subcores=NS)
  fn = pl.kernel(
      _sc_degree_body,
      out_type=jax.ShapeDtypeStruct((NC, NP), jnp.float32),
      mesh=mesh,
      scratch_types=[
          pltpu.VMEM((nch, CH), jnp.int32),
          pltpu.VMEM((CH,), jnp.float32),
          pltpu.VMEM((RPS,), jnp.float32),
          pltpu.VMEM_SHARED((NP,), jnp.float32),
          pltpu.SemaphoreType.DMA,
      ],
  )
  return fn(dst3)


def _sc_scatter(table, src3, dst3, nch):
  mesh = plsc.VectorSubcoreMesh(core_axis_name="c", subcore_axis_name="s",
                                num_cores=NC, num_subcores=NS)
  fn = pl.kernel(
      _sc_scatter_body,
      out_type=jax.ShapeDtypeStruct((NC, NP, F), jnp.float32),
      mesh=mesh,
      compiler_params=pltpu.CompilerParams(use_tc_tiling_on_sc=False),
      scratch_types=[
          pltpu.VMEM((nch, CH), jnp.int32),
          pltpu.VMEM((nch, CH), jnp.int32),
          pltpu.VMEM((4, CH, F), jnp.float32),
          pltpu.VMEM((ZR, F), jnp.float32),
          pltpu.VMEM_SHARED((NP, F), jnp.float32),
          [pltpu.SemaphoreType.DMA] * 4,
          [pltpu.SemaphoreType.DMA] * 4,
      ],
  )
  return fn(table, src3, dst3)


def _tc1_body(x_ref, w_ref, degp_ref, g_ref, dinv_ref):
  deg = degp_ref[0] + degp_ref[1] + 1.0
  dinv = lax.rsqrt(deg)
  h = jnp.dot(x_ref[...], w_ref[...], preferred_element_type=jnp.float32)
  dinv_ref[...] = dinv
  g_ref[...] = dinv * h


def _merge_partials(acc_ref):
  return acc_ref[0] + acc_ref[1]


def _tc2_body(acc_ref, g_ref, dinv_ref, b_ref, w_ref, g2_ref):
  dinv = dinv_ref[...]
  z = dinv * (_merge_partials(acc_ref) + g_ref[...]) + b_ref[...]
  o = jnp.maximum(z, 0.0)
  h2 = jnp.dot(o, w_ref[...], preferred_element_type=jnp.float32)
  g2_ref[...] = dinv * h2


def _tc3_body(acc_ref, g_ref, dinv_ref, b_ref, out_ref):
  z = dinv_ref[...] * (_merge_partials(acc_ref) + g_ref[...]) + b_ref[...]
  m = jnp.max(z, axis=1, keepdims=True)
  e = jnp.exp(z - m)
  lse = jnp.log(jnp.sum(e, axis=1, keepdims=True))
  out_ref[...] = z - m - lse


def _tc1(xp, W1, degp):
  grid = (NP // ROWBLK,)
  return pl.pallas_call(
      _tc1_body,
      grid=grid,
      in_specs=[
          pl.BlockSpec((ROWBLK, D), lambda i: (i, 0)),
          pl.BlockSpec((D, F), lambda i: (0, 0)),
          pl.BlockSpec((2, ROWBLK, 1), lambda i: (0, i, 0)),
      ],
      out_specs=[
          pl.BlockSpec((ROWBLK, F), lambda i: (i, 0)),
          pl.BlockSpec((ROWBLK, 1), lambda i: (i, 0)),
      ],
      out_shape=[
          jax.ShapeDtypeStruct((NP, F), jnp.float32),
          jax.ShapeDtypeStruct((NP, 1), jnp.float32),
      ],
  )(xp, W1, degp)


def _tc2(acc1, g1, dinv, b1, W2):
  grid = (NP // ROWBLK,)
  return pl.pallas_call(
      _tc2_body,
      grid=grid,
      in_specs=[
          pl.BlockSpec((NC, ROWBLK, F), lambda i: (0, i, 0)),
          pl.BlockSpec((ROWBLK, F), lambda i: (i, 0)),
          pl.BlockSpec((ROWBLK, 1), lambda i: (i, 0)),
          pl.BlockSpec((F,), lambda i: (0,)),
          pl.BlockSpec((F, F), lambda i: (0, 0)),
      ],
      out_specs=pl.BlockSpec((ROWBLK, F), lambda i: (i, 0)),
      out_shape=jax.ShapeDtypeStruct((NP, F), jnp.float32),
  )(acc1, g1, dinv, b1, W2)


def _tc3(acc2, g2, dinv, b2):
  grid = (NP // ROWBLK,)
  return pl.pallas_call(
      _tc3_body,
      grid=grid,
      in_specs=[
          pl.BlockSpec((NC, ROWBLK, F), lambda i: (0, i, 0)),
          pl.BlockSpec((ROWBLK, F), lambda i: (i, 0)),
          pl.BlockSpec((ROWBLK, 1), lambda i: (i, 0)),
          pl.BlockSpec((F,), lambda i: (0,)),
      ],
      out_specs=pl.BlockSpec((ROWBLK, F), lambda i: (i, 0)),
      out_shape=jax.ShapeDtypeStruct((NP, F), jnp.float32),
  )(acc2, g2, dinv, b2)


def kernel(x, edge_index, W1, b1, W2, b2):
  src = edge_index[0].astype(jnp.int32)
  dst = edge_index[1].astype(jnp.int32)
  e = src.shape[0]
  per = NW * CH
  nch = 4 * (-(-e // (4 * per)))
  ep = nch * per
  src_p = jnp.concatenate([src, jnp.zeros((ep - e,), jnp.int32)])
  dst_p = jnp.concatenate([dst, jnp.full((ep - e,), N, jnp.int32)])
  src3 = src_p.reshape(NW, nch, CH)
  dst3 = dst_p.reshape(NW, nch, CH)

  xp = jnp.pad(x.astype(jnp.float32), ((0, NP - N), (0, 0)))
  degp = _sc_degree(dst3, nch)
  g1, dinv = _tc1(xp, W1, degp.reshape(NC, NP, 1))
  acc1 = _sc_scatter(g1, src3, dst3, nch)
  g2 = _tc2(acc1, g1, dinv, b1, W2)
  acc2 = _sc_scatter(g2, src3, dst3, nch)
  out = _tc3(acc2, g2, dinv, b2)
  return out[:N]
